# baseline (speedup 1.0000x reference)
import functools

import jax
import jax.numpy as jnp
from jax import lax
from jax.experimental import pallas as pl
from jax.experimental.pallas import tpu as pltpu
from jax.experimental.pallas import tpu_sc as plsc

N_NODES = 10000
D = 128
F32 = jnp.float32


def _pre_body(h_ref, wl_ref, bl_ref, wr_ref, br_ref, xl_o, xr_o):
    h = h_ref[...]
    xl_o[...] = jnp.dot(h, wl_ref[...], preferred_element_type=F32) + bl_ref[...]
    xr_o[...] = jnp.dot(h, wr_ref[...], preferred_element_type=F32) + br_ref[...]


def _pre(h, wl, bl, wr, br):
    return pl.pallas_call(
        _pre_body,
        out_shape=[jax.ShapeDtypeStruct((N_NODES, D), F32)] * 2,
    )(h, wl, bl, wr, br)


def _combine_body(num_ref, den_ref, bias_ref, gamma_ref, beta_ref,
                  wl_ref, bl_ref, wr_ref, br_ref, xl_o, xr_o):
    num = num_ref[:N_NODES, :]
    den = den_ref[:N_NODES, 0:1]
    h = num / (den + 1e-16) + bias_ref[...]
    h = jnp.tanh(h)
    mu = jnp.mean(h)
    hc = h - mu
    sd = jnp.sqrt(jnp.mean(hc * hc))
    h = hc / (sd + 1e-5)
    h = h * gamma_ref[...] + beta_ref[...]
    xl_o[...] = jnp.dot(h, wl_ref[...], preferred_element_type=F32) + bl_ref[...]
    xr_o[...] = jnp.dot(h, wr_ref[...], preferred_element_type=F32) + br_ref[...]


def _combine(num2, den2, bias, gamma, beta, wl, bl, wr, br):
    return pl.pallas_call(
        _combine_body,
        out_shape=[jax.ShapeDtypeStruct((N_NODES, D), F32)] * 2,
    )(num2, den2, bias, gamma, beta, wl, bl, wr, br)


def _final_body(num_ref, den_ref, bias_ref, out_o):
    num = num_ref[:N_NODES, :]
    den = den_ref[:N_NODES, 0:1]
    out_o[...] = num / (den + 1e-16) + bias_ref[...]


def _final(num2, den2, bias):
    return pl.pallas_call(
        _final_body,
        out_shape=jax.ShapeDtypeStruct((N_NODES, D), F32),
    )(num2, den2, bias)


NW = 16
EPW = 20064
CHUNK = 32
CPAD = 32
NCHUNK = EPW // CHUNK
NB = 3
N_PAD = 10240
RPT = N_PAD // 16


def _sc_edge_body(xl_hbm, xr_hbm, edata, srcc, dstc, wv_hbm, av_hbm,
                  onum, oden,
                  idx_v, s0, s1, s2, d0, d1, d2, g0, g1, g2, r0, r1, r2,
                  den_l, w_v, a_v,
                  num_s,
                  si0, si1, si2, sg0, sg1, sg2, ss0, ss1, ss2):
    sid = lax.axis_index("s")
    wid = sid
    gxl = (g0, g1, g2)
    gxr = (r0, r1, r2)
    srcb = (s0, s1, s2)
    dst = (d0, d1, d2)
    sem_i = (si0, si1, si2)
    sem_g = (sg0, sg1, sg2)
    sem_s = (ss0, ss1, ss2)

    pltpu.sync_copy(wv_hbm, w_v)
    pltpu.sync_copy(av_hbm, a_v)

    zeros16 = jnp.zeros((16,), F32)
    lane = lax.broadcasted_iota(jnp.int32, (16,), 0)

    def zrow(r, carry):
        for j in range(8):
            g0[r, pl.ds(16 * j, 16)] = zeros16
        return carry

    lax.fori_loop(0, CHUNK, zrow, 0)

    def zden(r, carry):
        for j in range(8):
            den_l[r, pl.ds(16 * j, 16)] = zeros16
        return carry

    lax.fori_loop(0, N_PAD // 128, zden, 0)

    for k in range(RPT // CHUNK):
        pltpu.sync_copy(g0, num_s.at[pl.ds(sid * RPT + k * CHUNK, CHUNK)])
    plsc.subcore_barrier()

    def issue_idx(c, u):
        pltpu.async_copy(edata.at[wid, c], idx_v.at[u], sem_i[u])
        pltpu.async_copy(srcc.at[wid, c], srcb[u], sem_i[u])
        pltpu.async_copy(dstc.at[wid, c], dst[u], sem_i[u])

    def wait_idx(u):
        pltpu.make_async_copy(edata.at[0, 0], idx_v.at[u], sem_i[u]).wait()
        pltpu.make_async_copy(srcc.at[0, 0], srcb[u], sem_i[u]).wait()
        pltpu.make_async_copy(dstc.at[0, 0], dst[u], sem_i[u]).wait()

    def issue_gather(c, u):
        pltpu.async_copy(xl_hbm.at[srcb[u]], gxl[u], sem_g[u])
        pltpu.async_copy(xr_hbm.at[dst[u]], gxr[u], sem_g[u])

    def wait_gather(u):
        pltpu.make_async_copy(onum.at[pl.ds(0, CHUNK)], gxl[u], sem_g[u]).wait()
        pltpu.make_async_copy(onum.at[pl.ds(0, CHUNK)], gxr[u], sem_g[u]).wait()

    def issue_scatter(u):
        pltpu.async_copy(gxl[u], num_s.at[dst[u]], sem_s[u], add=True)

    def wait_scatter(u):
        pltpu.make_async_copy(gxl[u], num_s.at[pl.ds(0, CHUNK)], sem_s[u]).wait()

    def compute(u):
        ga, gb = gxl[u], gxr[u]
        for g, glen in ((0, 16), (1, 16)):
            gbase = g * 16
            dvv = idx_v[u, 0, pl.ds(gbase, 16)]
            evv = plsc.bitcast(idx_v[u, 1, pl.ds(gbase, 16)], F32)
            for k in range(glen):
                i = gbase + k
                ev = evv[k]
                dv = dvv[k]
                acc = zeros16
                asl = []
                for j in range(8):
                    av = ga[i, pl.ds(16 * j, 16)]
                    bv = gb[i, pl.ds(16 * j, 16)]
                    s = av + bv + ev * w_v[pl.ds(16 * j, 16)]
                    m = jnp.maximum(s, 0.2 * s)
                    acc = acc + m * a_v[pl.ds(16 * j, 16)]
                    asl.append(av)
                tot = jnp.sum(acc)
                exv = jnp.exp(jnp.full((16,), tot, F32))
                for j in range(8):
                    ga[i, pl.ds(16 * j, 16)] = exv * asl[j]
                dr = dv >> 7
                dcol = ((dv >> 4) & 7) * 16
                dl = den_l[dr, pl.ds(dcol, 16)]
                den_l[dr, pl.ds(dcol, 16)] = dl + jnp.where(
                    lane == (dv & 15), exv, 0.0)

    for u in range(NB):
        pltpu.sync_copy(edata.at[wid, u], idx_v.at[u])
        pltpu.sync_copy(srcc.at[wid, u], srcb[u])
        pltpu.sync_copy(dstc.at[wid, u], dst[u])
    issue_gather(0, 0)
    issue_gather(1, 1)

    def tri_body(t, carry):
        for u in range(NB):
            c = t * NB + u
            wait_gather(u)
            compute(u)
            issue_scatter(u)

            @pl.when(c + NB <= NCHUNK - 1)
            def _():
                issue_idx(c + NB, u)

            up2 = (u + 2) % NB

            @pl.when(jnp.logical_and(c >= 1, c + 2 <= NCHUNK - 1))
            def _():
                wait_idx(up2)

            @pl.when(jnp.logical_and(c >= 1, c + 2 <= NCHUNK - 1))
            def _():
                wait_scatter(up2)

            @pl.when(c + 2 <= NCHUNK - 1)
            def _():
                issue_gather(c + 2, up2)
        return carry

    lax.fori_loop(0, NCHUNK // NB, tri_body, 0)

    wait_scatter(0)
    wait_scatter(1)
    wait_scatter(2)
    plsc.subcore_barrier()

    for k in range(RPT // CHUNK):
        r0_ = sid * RPT + k * CHUNK
        pltpu.sync_copy(num_s.at[pl.ds(r0_, CHUNK)], g0)
        pltpu.sync_copy(g0, onum.at[pl.ds(r0_, CHUNK)])
    pltpu.sync_copy(den_l, oden.at[wid])


@functools.partial(
    pl.kernel,
    out_type=[jax.ShapeDtypeStruct((N_PAD, D), F32),
              jax.ShapeDtypeStruct((NW, N_PAD // 128, D), F32)],
    mesh=plsc.VectorSubcoreMesh(core_axis_name="c", subcore_axis_name="s",
                                num_cores=1),
    compiler_params=pltpu.CompilerParams(needs_layout_passes=False),
    scratch_types=[
        pltpu.VMEM((NB, 2, 128), jnp.int32),
        pltpu.VMEM((CHUNK,), jnp.int32),
        pltpu.VMEM((CHUNK,), jnp.int32),
        pltpu.VMEM((CHUNK,), jnp.int32),
        pltpu.VMEM((CHUNK,), jnp.int32),
        pltpu.VMEM((CHUNK,), jnp.int32),
        pltpu.VMEM((CHUNK,), jnp.int32),
        pltpu.VMEM((CHUNK, D), F32),
        pltpu.VMEM((CHUNK, D), F32),
        pltpu.VMEM((CHUNK, D), F32),
        pltpu.VMEM((CHUNK, D), F32),
        pltpu.VMEM((CHUNK, D), F32),
        pltpu.VMEM((CHUNK, D), F32),
        pltpu.VMEM((N_PAD // 128, D), F32),
        pltpu.VMEM((D,), F32),
        pltpu.VMEM((D,), F32),
        pltpu.VMEM_SHARED((N_PAD, D), F32),
        pltpu.SemaphoreType.DMA,
        pltpu.SemaphoreType.DMA,
        pltpu.SemaphoreType.DMA,
        pltpu.SemaphoreType.DMA,
        pltpu.SemaphoreType.DMA,
        pltpu.SemaphoreType.DMA,
        pltpu.SemaphoreType.DMA,
        pltpu.SemaphoreType.DMA,
        pltpu.SemaphoreType.DMA,
    ],
)
def _sc_edge(xl, xr, edata, srcc, dstc, wv, av, *rest):
    _sc_edge_body(xl, xr, edata, srcc, dstc, wv, av, *rest)


def kernel(x, e, params, edge_index):
    npad = EPW - 320000 // NW
    srcw = jnp.concatenate(
        [edge_index[0].reshape(NW, -1),
         jnp.zeros((NW, npad), jnp.int32)], axis=1)
    dstw = jnp.concatenate(
        [edge_index[1].reshape(NW, -1),
         jnp.full((NW, npad), N_NODES, jnp.int32)], axis=1)
    ew = jnp.concatenate(
        [lax.bitcast_convert_type(e[:, 0], jnp.int32).reshape(NW, -1),
         jnp.zeros((NW, npad), jnp.int32)], axis=1)
    rowpad = jnp.zeros((NW, NCHUNK, 128 - CHUNK), jnp.int32)
    edata = jnp.stack(
        [jnp.concatenate([a.reshape(NW, NCHUNK, CHUNK), rowpad], axis=2)
         for a in (dstw, ew)], axis=2)
    srcc = srcw.reshape(NW, NCHUNK, CHUNK)
    dstc = dstw.reshape(NW, NCHUNK, CHUNK)
    gat = params['gat']
    norm = params['norm']

    def r2(v):
        return v.reshape(1, D)

    h = x
    p = gat[0]
    xl, xr = _pre(h, p['Wl'], r2(p['bl']), p['Wr'], r2(p['br']))
    for i in range(4):
        p = gat[i]
        we = p['We'][0]
        num2, dpart = _sc_edge(xl, xr, edata, srcc, dstc, we, p['att'])
        den2 = dpart.sum(axis=0).reshape(N_PAD, 1)
        if i < 3:
            q = gat[i + 1]
            xl, xr = _combine(num2, den2, r2(p['bias']),
                              r2(norm[i]['gamma']), r2(norm[i]['beta']),
                              q['Wl'], r2(q['bl']), q['Wr'], r2(q['br']))
        else:
            return _final(num2, den2, r2(p['bias']))


# speedup vs baseline: 2.4540x; 2.4540x over previous
import functools

import jax
import jax.numpy as jnp
from jax import lax
from jax.experimental import pallas as pl
from jax.experimental.pallas import tpu as pltpu
from jax.experimental.pallas import tpu_sc as plsc

N_NODES = 10000
D = 128
F32 = jnp.float32


def _pre_body(h_ref, wl_ref, bl_ref, wr_ref, br_ref, xl_o, xr_o):
    h = h_ref[...]
    xl_o[...] = jnp.dot(h, wl_ref[...], preferred_element_type=F32) + bl_ref[...]
    xr_o[...] = jnp.dot(h, wr_ref[...], preferred_element_type=F32) + br_ref[...]


def _pre(h, wl, bl, wr, br):
    return pl.pallas_call(
        _pre_body,
        out_shape=[jax.ShapeDtypeStruct((N_NODES, D), F32)] * 2,
    )(h, wl, bl, wr, br)


def _combine_body(num_ref, den_ref, bias_ref, gamma_ref, beta_ref,
                  wl_ref, bl_ref, wr_ref, br_ref, xl_o, xr_o):
    num = num_ref[:N_NODES, :]
    den = den_ref[:N_NODES, 0:1]
    h = num / (den + 1e-16) + bias_ref[...]
    h = jnp.tanh(h)
    mu = jnp.mean(h)
    hc = h - mu
    sd = jnp.sqrt(jnp.mean(hc * hc))
    h = hc / (sd + 1e-5)
    h = h * gamma_ref[...] + beta_ref[...]
    xl_o[...] = jnp.dot(h, wl_ref[...], preferred_element_type=F32) + bl_ref[...]
    xr_o[...] = jnp.dot(h, wr_ref[...], preferred_element_type=F32) + br_ref[...]


def _combine(num2, den2, bias, gamma, beta, wl, bl, wr, br):
    return pl.pallas_call(
        _combine_body,
        out_shape=[jax.ShapeDtypeStruct((N_NODES, D), F32)] * 2,
    )(num2, den2, bias, gamma, beta, wl, bl, wr, br)


def _final_body(num_ref, den_ref, bias_ref, out_o):
    num = num_ref[:N_NODES, :]
    den = den_ref[:N_NODES, 0:1]
    out_o[...] = num / (den + 1e-16) + bias_ref[...]


def _final(num2, den2, bias):
    return pl.pallas_call(
        _final_body,
        out_shape=jax.ShapeDtypeStruct((N_NODES, D), F32),
    )(num2, den2, bias)


NW = 16
EPW = 20064
CHUNK = 32
CPAD = 32
NCHUNK = EPW // CHUNK
NB = 3
N_PAD = 10240
RPT = N_PAD // 16


def _sc_edge_body(xl_hbm, xr_hbm, edata, srcc, dstc, wv_hbm, av_hbm,
                  onum, oden,
                  idx_v, s0, s1, s2, d0, d1, d2, g0, g1, g2, r0, r1, r2,
                  den_l, w_v, a_v,
                  num_s,
                  si0, si1, si2, sg0, sg1, sg2, ss0, ss1, ss2):
    sid = lax.axis_index("s")
    wid = sid
    gxl = (g0, g1, g2)
    gxr = (r0, r1, r2)
    srcb = (s0, s1, s2)
    dst = (d0, d1, d2)
    sem_i = (si0, si1, si2)
    sem_g = (sg0, sg1, sg2)
    sem_s = (ss0, ss1, ss2)

    pltpu.sync_copy(wv_hbm, w_v)
    pltpu.sync_copy(av_hbm, a_v)

    zeros16 = jnp.zeros((16,), F32)
    lane = lax.broadcasted_iota(jnp.int32, (16,), 0)

    def zrow(r, carry):
        for j in range(8):
            g0[r, pl.ds(16 * j, 16)] = zeros16
        return carry

    lax.fori_loop(0, CHUNK, zrow, 0)

    def zden(r, carry):
        for j in range(8):
            den_l[r, pl.ds(16 * j, 16)] = zeros16
        return carry

    lax.fori_loop(0, N_PAD // 128, zden, 0)

    for k in range(RPT // CHUNK):
        pltpu.sync_copy(g0, num_s.at[pl.ds(sid * RPT + k * CHUNK, CHUNK)])
    plsc.subcore_barrier()

    def issue_idx(c, u):
        pltpu.async_copy(edata.at[wid, c], idx_v.at[u], sem_i[u])
        pltpu.async_copy(srcc.at[wid, c], srcb[u], sem_i[u])
        pltpu.async_copy(dstc.at[wid, c], dst[u], sem_i[u])

    def wait_idx(u):
        pltpu.make_async_copy(edata.at[0, 0], idx_v.at[u], sem_i[u]).wait()
        pltpu.make_async_copy(srcc.at[0, 0], srcb[u], sem_i[u]).wait()
        pltpu.make_async_copy(dstc.at[0, 0], dst[u], sem_i[u]).wait()

    def issue_gather(c, u):
        pltpu.async_copy(xl_hbm.at[srcb[u]], gxl[u], sem_g[u])
        pltpu.async_copy(xr_hbm.at[dst[u]], gxr[u], sem_g[u])

    def wait_gather(u):
        pltpu.make_async_copy(onum.at[pl.ds(0, CHUNK)], gxl[u], sem_g[u]).wait()
        pltpu.make_async_copy(onum.at[pl.ds(0, CHUNK)], gxr[u], sem_g[u]).wait()

    def issue_scatter(u):
        pltpu.async_copy(gxl[u], num_s.at[dst[u]], sem_s[u], add=True)

    def wait_scatter(u):
        pltpu.make_async_copy(gxl[u], num_s.at[pl.ds(0, CHUNK)], sem_s[u]).wait()

    wv8 = [w_v[pl.ds(16 * j, 16)] for j in range(8)]
    av8 = [a_v[pl.ds(16 * j, 16)] for j in range(8)]

    def compute(u):
        ga, gb = gxl[u], gxr[u]
        for g in range(CHUNK // 16):
            gbase = g * 16
            dvv = idx_v[u, 0, pl.ds(gbase, 16)]
            evv = plsc.bitcast(idx_v[u, 1, pl.ds(gbase, 16)], F32)
            exg = zeros16
            for k in range(16):
                i = gbase + k
                ev = evv[k]
                acc = zeros16
                asl = []
                for j in range(8):
                    av = ga[i, pl.ds(16 * j, 16)]
                    bv = gb[i, pl.ds(16 * j, 16)]
                    s = av + bv + ev * wv8[j]
                    m = jnp.maximum(s, 0.2 * s)
                    acc = acc + m * av8[j]
                    asl.append(av)
                tot = jnp.sum(acc)
                exv = jnp.exp(jnp.full((16,), tot, F32))
                for j in range(8):
                    ga[i, pl.ds(16 * j, 16)] = exv * asl[j]
                exg = jnp.where(lane == k, exv, exg)
            plsc.addupdate_scatter(den_l, [dvv >> 7, dvv & 127], exg)

    for u in range(NB):
        pltpu.sync_copy(edata.at[wid, u], idx_v.at[u])
        pltpu.sync_copy(srcc.at[wid, u], srcb[u])
        pltpu.sync_copy(dstc.at[wid, u], dst[u])
    issue_gather(0, 0)
    issue_gather(1, 1)

    def tri_body(t, carry):
        for u in range(NB):
            c = t * NB + u
            wait_gather(u)
            compute(u)
            issue_scatter(u)

            @pl.when(c + NB <= NCHUNK - 1)
            def _():
                issue_idx(c + NB, u)

            up2 = (u + 2) % NB

            @pl.when(jnp.logical_and(c >= 1, c + 2 <= NCHUNK - 1))
            def _():
                wait_idx(up2)

            @pl.when(jnp.logical_and(c >= 1, c + 2 <= NCHUNK - 1))
            def _():
                wait_scatter(up2)

            @pl.when(c + 2 <= NCHUNK - 1)
            def _():
                issue_gather(c + 2, up2)
        return carry

    lax.fori_loop(0, NCHUNK // NB, tri_body, 0)

    wait_scatter(0)
    wait_scatter(1)
    wait_scatter(2)
    plsc.subcore_barrier()

    for k in range(RPT // CHUNK):
        r0_ = sid * RPT + k * CHUNK
        pltpu.sync_copy(num_s.at[pl.ds(r0_, CHUNK)], g0)
        pltpu.sync_copy(g0, onum.at[pl.ds(r0_, CHUNK)])
    pltpu.sync_copy(den_l, oden.at[wid])


@functools.partial(
    pl.kernel,
    out_type=[jax.ShapeDtypeStruct((N_PAD, D), F32),
              jax.ShapeDtypeStruct((NW, N_PAD // 128, D), F32)],
    mesh=plsc.VectorSubcoreMesh(core_axis_name="c", subcore_axis_name="s",
                                num_cores=1),
    compiler_params=pltpu.CompilerParams(needs_layout_passes=False),
    scratch_types=[
        pltpu.VMEM((NB, 2, 128), jnp.int32),
        pltpu.VMEM((CHUNK,), jnp.int32),
        pltpu.VMEM((CHUNK,), jnp.int32),
        pltpu.VMEM((CHUNK,), jnp.int32),
        pltpu.VMEM((CHUNK,), jnp.int32),
        pltpu.VMEM((CHUNK,), jnp.int32),
        pltpu.VMEM((CHUNK,), jnp.int32),
        pltpu.VMEM((CHUNK, D), F32),
        pltpu.VMEM((CHUNK, D), F32),
        pltpu.VMEM((CHUNK, D), F32),
        pltpu.VMEM((CHUNK, D), F32),
        pltpu.VMEM((CHUNK, D), F32),
        pltpu.VMEM((CHUNK, D), F32),
        pltpu.VMEM((N_PAD // 128, D), F32),
        pltpu.VMEM((D,), F32),
        pltpu.VMEM((D,), F32),
        pltpu.VMEM_SHARED((N_PAD, D), F32),
        pltpu.SemaphoreType.DMA,
        pltpu.SemaphoreType.DMA,
        pltpu.SemaphoreType.DMA,
        pltpu.SemaphoreType.DMA,
        pltpu.SemaphoreType.DMA,
        pltpu.SemaphoreType.DMA,
        pltpu.SemaphoreType.DMA,
        pltpu.SemaphoreType.DMA,
        pltpu.SemaphoreType.DMA,
    ],
)
def _sc_edge(xl, xr, edata, srcc, dstc, wv, av, *rest):
    _sc_edge_body(xl, xr, edata, srcc, dstc, wv, av, *rest)


def kernel(x, e, params, edge_index):
    npad = EPW - 320000 // NW
    srcw = jnp.concatenate(
        [edge_index[0].reshape(NW, -1),
         jnp.zeros((NW, npad), jnp.int32)], axis=1)
    dstw = jnp.concatenate(
        [edge_index[1].reshape(NW, -1),
         jnp.full((NW, npad), N_NODES, jnp.int32)], axis=1)
    ew = jnp.concatenate(
        [lax.bitcast_convert_type(e[:, 0], jnp.int32).reshape(NW, -1),
         jnp.zeros((NW, npad), jnp.int32)], axis=1)
    rowpad = jnp.zeros((NW, NCHUNK, 128 - CHUNK), jnp.int32)
    edata = jnp.stack(
        [jnp.concatenate([a.reshape(NW, NCHUNK, CHUNK), rowpad], axis=2)
         for a in (dstw, ew)], axis=2)
    srcc = srcw.reshape(NW, NCHUNK, CHUNK)
    dstc = dstw.reshape(NW, NCHUNK, CHUNK)
    gat = params['gat']
    norm = params['norm']

    def r2(v):
        return v.reshape(1, D)

    h = x
    p = gat[0]
    xl, xr = _pre(h, p['Wl'], r2(p['bl']), p['Wr'], r2(p['br']))
    for i in range(4):
        p = gat[i]
        we = p['We'][0]
        num2, dpart = _sc_edge(xl, xr, edata, srcc, dstc, we, p['att'])
        den2 = dpart.sum(axis=0).reshape(N_PAD, 1)
        if i < 3:
            q = gat[i + 1]
            xl, xr = _combine(num2, den2, r2(p['bias']),
                              r2(norm[i]['gamma']), r2(norm[i]['beta']),
                              q['Wl'], r2(q['bl']), q['Wr'], r2(q['br']))
        else:
            return _final(num2, den2, r2(p['bias']))
